# baseline (speedup 1.0000x reference)
import functools
import math

import jax
import jax.numpy as jnp
from jax import lax
from jax.experimental import pallas as pl
from jax.experimental.pallas import tpu as pltpu
from jax.experimental.pallas import tpu_sc as plsc

BINS = 10
B = 16384
C = 1000
N = B * C
NC, NS, LANES = 2, 16, 16
NW = NC * NS
PERW = N // NW
CHE = 16_000
NCHUNK = PERW // CHE
VPC = CHE // LANES
ACCW = 2 * BINS * LANES

_THRESH = [math.log(k / (BINS - k)) for k in range(1, BINS)]

_L1P = [1.4720650114430356e-06, 0.9998476974962336, -0.497373216157962,
        0.31574731675807266, -0.19035433673330746, 0.08269123711164639,
        -0.01741407752433588]

_mesh = plsc.VectorSubcoreMesh(core_axis_name="c", subcore_axis_name="s")


@functools.partial(
    pl.kernel,
    out_type=jax.ShapeDtypeStruct((NW * ACCW,), jnp.float32),
    mesh=_mesh,
    scratch_types=[
        pltpu.VMEM((CHE,), jnp.float32),
        pltpu.VMEM((CHE,), jnp.float32),
        pltpu.VMEM((ACCW,), jnp.float32),
    ],
    compiler_params=pltpu.CompilerParams(needs_layout_passes=False),
)
def _sc_main(x_hbm, t_hbm, out_hbm, xb, tb, acc):
    wid = lax.axis_index("s") * NC + lax.axis_index("c")
    base = wid * PERW

    zeros = jnp.zeros((LANES,), jnp.float32)
    ones = jnp.ones((LANES,), jnp.float32)
    lane = lax.iota(jnp.int32, LANES)
    for k in range(2 * BINS):
        acc[pl.ds(k * LANES, LANES)] = zeros

    def chunk_body(ci, carry):
        off = base + ci * CHE
        pltpu.sync_copy(x_hbm.at[pl.ds(off, CHE)], xb)
        pltpu.sync_copy(t_hbm.at[pl.ds(off, CHE)], tb)

        def vec_body(j, carry2):
            x = xb[pl.ds(j * LANES, LANES)]
            t = tb[pl.ds(j * LANES, LANES)]
            y = jnp.where(t > 0.5, -x, x)
            z = jnp.abs(y)
            u = jnp.exp(-z)
            p = jnp.float32(_L1P[6])
            for c in (5, 4, 3, 2, 1, 0):
                p = p * u + jnp.float32(_L1P[c])
            bce = jnp.maximum(y, 0.0) + p
            binf = zeros
            for k in range(BINS - 1):
                binf = binf + jnp.where(y >= _THRESH[k], 1.0, 0.0)
            idx = binf.astype(jnp.int32) * LANES + lane
            plsc.addupdate_scatter(acc, [idx], bce)
            plsc.addupdate_scatter(acc, [idx + BINS * LANES], ones)
            return carry2

        return lax.fori_loop(0, VPC, vec_body, carry)

    lax.fori_loop(0, NCHUNK, chunk_body, 0)
    pltpu.sync_copy(acc, out_hbm.at[pl.ds(wid * ACCW, ACCW)])


def _tc_combine_body(a_ref, out_ref):
    a = a_ref[...]
    n = jnp.float32(0.0)
    s = jnp.float32(0.0)
    for b in range(BINS):
        sb = jnp.sum(a[:, 16 * b:16 * b + 16])
        cnt = jnp.sum(a[:, 160 + 16 * b:160 + 16 * b + 16])
        nonempty = cnt > 0.0
        n += jnp.where(nonempty, 1.0, 0.0)
        s += jnp.where(nonempty, sb / jnp.maximum(cnt, 1.0), 0.0)
    out_ref[0] = s / jnp.maximum(n, 1.0)


@jax.jit
def kernel(input, target):
    accs = _sc_main(input.reshape(-1), target.reshape(-1))
    out = pl.pallas_call(
        _tc_combine_body,
        out_specs=pl.BlockSpec(memory_space=pltpu.SMEM),
        out_shape=jax.ShapeDtypeStruct((1,), jnp.float32),
    )(accs.reshape(NW, ACCW))
    return out[0]


# speedup vs baseline: 1.0403x; 1.0403x over previous
import functools
import math

import jax
import jax.numpy as jnp
from jax import lax
from jax.experimental import pallas as pl
from jax.experimental.pallas import tpu as pltpu
from jax.experimental.pallas import tpu_sc as plsc

BINS = 10
B = 16384
C = 1000
N = B * C
NC, NS, LANES = 2, 16, 16
NW = NC * NS
PERW = N // NW
CHE = 16_000
NCHUNK = PERW // CHE
NPAIR = NCHUNK // 2
VPC = CHE // LANES
UNROLL = 4
ACCW = 2 * BINS * LANES

_L1P = [9.975032552456131e-06, 0.999235483833275, -0.4902307234234118,
        0.2852726810905759, -0.13158182508875882, 0.030449004538667168]

_mesh = plsc.VectorSubcoreMesh(core_axis_name="c", subcore_axis_name="s")


@functools.partial(
    pl.kernel,
    out_type=jax.ShapeDtypeStruct((NW * ACCW,), jnp.float32),
    mesh=_mesh,
    scratch_types=[
        pltpu.VMEM((CHE,), jnp.float32),
        pltpu.VMEM((CHE,), jnp.float32),
        pltpu.VMEM((CHE,), jnp.float32),
        pltpu.VMEM((CHE,), jnp.float32),
        pltpu.VMEM((ACCW,), jnp.float32),
        pltpu.SemaphoreType.DMA,
        pltpu.SemaphoreType.DMA,
        pltpu.SemaphoreType.DMA,
        pltpu.SemaphoreType.DMA,
    ],
    compiler_params=pltpu.CompilerParams(needs_layout_passes=False),
)
def _sc_main(x_hbm, t_hbm, out_hbm, xb0, tb0, xb1, tb1, acc,
             sx0, st0, sx1, st1):
    wid = lax.axis_index("s") * NC + lax.axis_index("c")
    base = wid * PERW

    zeros = jnp.zeros((LANES,), jnp.float32)
    ones = jnp.ones((LANES,), jnp.float32)
    lane = lax.iota(jnp.int32, LANES)
    for k in range(2 * BINS):
        acc[pl.ds(k * LANES, LANES)] = zeros

    def start(off, xb, tb, sx, st):
        pltpu.make_async_copy(x_hbm.at[pl.ds(off, CHE)], xb, sx).start()
        pltpu.make_async_copy(t_hbm.at[pl.ds(off, CHE)], tb, st).start()

    def wait(off, xb, tb, sx, st):
        pltpu.make_async_copy(x_hbm.at[pl.ds(off, CHE)], xb, sx).wait()
        pltpu.make_async_copy(t_hbm.at[pl.ds(off, CHE)], tb, st).wait()

    def compute(xb, tb):
        def vec_body(j, carry):
            for k4 in range(UNROLL):
                o = j * (LANES * UNROLL) + k4 * LANES
                x = xb[pl.ds(o, LANES)]
                t = tb[pl.ds(o, LANES)]
                y = jnp.where(t > 0.5, -x, x)
                u = jnp.exp(-jnp.abs(y))
                p = jnp.float32(_L1P[5])
                for c in (4, 3, 2, 1, 0):
                    p = p * u + jnp.float32(_L1P[c])
                bce = jnp.maximum(y, 0.0) + p
                sig = jnp.where(y >= 0.0, 1.0, u) / (1.0 + u)
                bidx = jnp.minimum((sig * BINS).astype(jnp.int32), BINS - 1)
                idx = bidx * LANES + lane
                plsc.addupdate_scatter(acc, [idx], bce)
                plsc.addupdate_scatter(acc, [idx + BINS * LANES], ones)
            return carry

        lax.fori_loop(0, VPC // UNROLL, vec_body, 0)

    start(base, xb0, tb0, sx0, st0)

    def pair_body(i, carry):
        off0 = base + (2 * i) * CHE
        off1 = off0 + CHE
        start(off1, xb1, tb1, sx1, st1)
        wait(off0, xb0, tb0, sx0, st0)
        compute(xb0, tb0)

        @pl.when(i < NPAIR - 1)
        def _prefetch_even():
            start(off1 + CHE, xb0, tb0, sx0, st0)

        wait(off1, xb1, tb1, sx1, st1)
        compute(xb1, tb1)
        return carry

    lax.fori_loop(0, NPAIR, pair_body, 0)
    pltpu.sync_copy(acc, out_hbm.at[pl.ds(wid * ACCW, ACCW)])


def _tc_combine_body(a_ref, out_ref):
    a = a_ref[...]
    n = jnp.float32(0.0)
    s = jnp.float32(0.0)
    for b in range(BINS):
        sb = jnp.sum(a[:, 16 * b:16 * b + 16])
        cnt = jnp.sum(a[:, 160 + 16 * b:160 + 16 * b + 16])
        nonempty = cnt > 0.0
        n += jnp.where(nonempty, 1.0, 0.0)
        s += jnp.where(nonempty, sb / jnp.maximum(cnt, 1.0), 0.0)
    out_ref[0] = s / jnp.maximum(n, 1.0)


@jax.jit
def kernel(input, target):
    accs = _sc_main(input.reshape(-1), target.reshape(-1))
    out = pl.pallas_call(
        _tc_combine_body,
        out_specs=pl.BlockSpec(memory_space=pltpu.SMEM),
        out_shape=jax.ShapeDtypeStruct((1,), jnp.float32),
    )(accs.reshape(NW, ACCW))
    return out[0]


# speedup vs baseline: 2.8581x; 2.7474x over previous
import functools
import math

import jax
import jax.numpy as jnp
from jax import lax
from jax.experimental import pallas as pl
from jax.experimental.pallas import tpu as pltpu
from jax.experimental.pallas import tpu_sc as plsc

BINS = 10
B = 16384
C = 1000
N = B * C
NC, NS, LANES = 2, 16, 16
NW = NC * NS
PERW = N // NW
CHE = 16_000
NCHUNK = PERW // CHE
NPAIR = NCHUNK // 2
VPC = CHE // LANES
NREP = 4
ACCW = 2 * BINS * LANES

_L1P = [9.975032552456131e-06, 0.999235483833275, -0.4902307234234118,
        0.2852726810905759, -0.13158182508875882, 0.030449004538667168]

_mesh = plsc.VectorSubcoreMesh(core_axis_name="c", subcore_axis_name="s")


@functools.partial(
    pl.kernel,
    out_type=jax.ShapeDtypeStruct((NW * NREP * ACCW,), jnp.float32),
    mesh=_mesh,
    scratch_types=[
        pltpu.VMEM((CHE,), jnp.float32),
        pltpu.VMEM((CHE,), jnp.float32),
        pltpu.VMEM((CHE,), jnp.float32),
        pltpu.VMEM((CHE,), jnp.float32),
        pltpu.VMEM((NREP * ACCW,), jnp.float32),
        pltpu.SemaphoreType.DMA,
        pltpu.SemaphoreType.DMA,
        pltpu.SemaphoreType.DMA,
        pltpu.SemaphoreType.DMA,
    ],
    compiler_params=pltpu.CompilerParams(needs_layout_passes=False),
)
def _sc_main(x_hbm, t_hbm, out_hbm, xb0, tb0, xb1, tb1, acc,
             sx0, st0, sx1, st1):
    wid = lax.axis_index("s") * NC + lax.axis_index("c")
    base = wid * PERW

    zeros = jnp.zeros((LANES,), jnp.float32)
    ones = jnp.ones((LANES,), jnp.float32)
    lane = lax.iota(jnp.int32, LANES)
    for k in range(NREP * 2 * BINS):
        acc[pl.ds(k * LANES, LANES)] = zeros

    def start(off, xb, tb, sx, st):
        pltpu.make_async_copy(x_hbm.at[pl.ds(off, CHE)], xb, sx).start()
        pltpu.make_async_copy(t_hbm.at[pl.ds(off, CHE)], tb, st).start()

    def wait(off, xb, tb, sx, st):
        pltpu.make_async_copy(x_hbm.at[pl.ds(off, CHE)], xb, sx).wait()
        pltpu.make_async_copy(t_hbm.at[pl.ds(off, CHE)], tb, st).wait()

    def compute(xb, tb):
        @plsc.parallel_loop(0, VPC, 1, unroll=8)
        def _vec(j):
            o = j * LANES
            x = xb[pl.ds(o, LANES)]
            t = tb[pl.ds(o, LANES)]
            y = jnp.where(t > 0.5, -x, x)
            u = jnp.exp(-jnp.abs(y))
            p = jnp.float32(_L1P[5])
            for c in (4, 3, 2, 1, 0):
                p = p * u + jnp.float32(_L1P[c])
            bce = jnp.maximum(y, 0.0) + p
            sig = jnp.where(y >= 0.0, 1.0, u) / (1.0 + u)
            bidx = jnp.minimum((sig * BINS).astype(jnp.int32), BINS - 1)
            rep = (j % NREP) * ACCW
            idx = bidx * LANES + lane + rep
            plsc.addupdate_scatter(acc, [idx], bce)
            plsc.addupdate_scatter(acc, [idx + BINS * LANES], ones)

    start(base, xb0, tb0, sx0, st0)

    def pair_body(i, carry):
        off0 = base + (2 * i) * CHE
        off1 = off0 + CHE
        start(off1, xb1, tb1, sx1, st1)
        wait(off0, xb0, tb0, sx0, st0)
        compute(xb0, tb0)

        @pl.when(i < NPAIR - 1)
        def _prefetch_even():
            start(off1 + CHE, xb0, tb0, sx0, st0)

        wait(off1, xb1, tb1, sx1, st1)
        compute(xb1, tb1)
        return carry

    lax.fori_loop(0, NPAIR, pair_body, 0)
    pltpu.sync_copy(acc, out_hbm.at[pl.ds(wid * NREP * ACCW, NREP * ACCW)])


def _tc_combine_body(a_ref, out_ref):
    a = a_ref[...]
    n = jnp.float32(0.0)
    s = jnp.float32(0.0)
    for b in range(BINS):
        sb = jnp.sum(a[:, 16 * b:16 * b + 16])
        cnt = jnp.sum(a[:, 160 + 16 * b:160 + 16 * b + 16])
        nonempty = cnt > 0.0
        n += jnp.where(nonempty, 1.0, 0.0)
        s += jnp.where(nonempty, sb / jnp.maximum(cnt, 1.0), 0.0)
    out_ref[0] = s / jnp.maximum(n, 1.0)


@jax.jit
def kernel(input, target):
    accs = _sc_main(input.reshape(-1), target.reshape(-1))
    out = pl.pallas_call(
        _tc_combine_body,
        out_specs=pl.BlockSpec(memory_space=pltpu.SMEM),
        out_shape=jax.ShapeDtypeStruct((1,), jnp.float32),
    )(accs.reshape(NW * NREP, ACCW))
    return out[0]


# speedup vs baseline: 3.2991x; 1.1543x over previous
import functools
import math

import jax
import jax.numpy as jnp
from jax import lax
from jax.experimental import pallas as pl
from jax.experimental.pallas import tpu as pltpu
from jax.experimental.pallas import tpu_sc as plsc

BINS = 10
B = 16384
C = 1000

SC_ROWS = 8192
TC_ROWS = B - SC_ROWS

SCN = SC_ROWS * C
NC, NS, LANES = 2, 16, 16
NW = NC * NS
PERW = SCN // NW
CHE = 16_000
NCHUNK = PERW // CHE
NPAIR = NCHUNK // 2
VPC = CHE // LANES
NREP = 4
ACCW = 2 * BINS * LANES

BLK = 256
RCH = 8
NACC = 2 * (BINS - 1) + 1
_THRESH = [math.log(k / (BINS - k)) for k in range(1, BINS)]
_COLS = [(ci * 128, 128) for ci in range(7)] + [(896, 104)]

_L1P = [9.975032552456131e-06, 0.999235483833275, -0.4902307234234118,
        0.2852726810905759, -0.13158182508875882, 0.030449004538667168]

_mesh = plsc.VectorSubcoreMesh(core_axis_name="c", subcore_axis_name="s")


@functools.partial(
    pl.kernel,
    out_type=jax.ShapeDtypeStruct((NW * NREP * ACCW,), jnp.float32),
    mesh=_mesh,
    scratch_types=[
        pltpu.VMEM((CHE,), jnp.float32),
        pltpu.VMEM((CHE,), jnp.float32),
        pltpu.VMEM((CHE,), jnp.float32),
        pltpu.VMEM((CHE,), jnp.float32),
        pltpu.VMEM((NREP * ACCW,), jnp.float32),
        pltpu.SemaphoreType.DMA,
        pltpu.SemaphoreType.DMA,
        pltpu.SemaphoreType.DMA,
        pltpu.SemaphoreType.DMA,
    ],
    compiler_params=pltpu.CompilerParams(needs_layout_passes=False),
)
def _sc_main(x_hbm, t_hbm, out_hbm, xb0, tb0, xb1, tb1, acc,
             sx0, st0, sx1, st1):
    wid = lax.axis_index("s") * NC + lax.axis_index("c")
    base = wid * PERW

    zeros = jnp.zeros((LANES,), jnp.float32)
    ones = jnp.ones((LANES,), jnp.float32)
    lane = lax.iota(jnp.int32, LANES)
    for k in range(NREP * 2 * BINS):
        acc[pl.ds(k * LANES, LANES)] = zeros

    def start(off, xb, tb, sx, st):
        pltpu.make_async_copy(x_hbm.at[pl.ds(off, CHE)], xb, sx).start()
        pltpu.make_async_copy(t_hbm.at[pl.ds(off, CHE)], tb, st).start()

    def wait(off, xb, tb, sx, st):
        pltpu.make_async_copy(x_hbm.at[pl.ds(off, CHE)], xb, sx).wait()
        pltpu.make_async_copy(t_hbm.at[pl.ds(off, CHE)], tb, st).wait()

    def compute(xb, tb):
        @plsc.parallel_loop(0, VPC, 1, unroll=8)
        def _vec(j):
            o = j * LANES
            x = xb[pl.ds(o, LANES)]
            t = tb[pl.ds(o, LANES)]
            y = jnp.where(t > 0.5, -x, x)
            u = jnp.exp(-jnp.abs(y))
            p = jnp.float32(_L1P[5])
            for c in (4, 3, 2, 1, 0):
                p = p * u + jnp.float32(_L1P[c])
            bce = jnp.maximum(y, 0.0) + p
            sig = jnp.where(y >= 0.0, 1.0, u) / (1.0 + u)
            bidx = jnp.minimum((sig * BINS).astype(jnp.int32), BINS - 1)
            rep = (j % NREP) * ACCW
            idx = bidx * LANES + lane + rep
            plsc.addupdate_scatter(acc, [idx], bce)
            plsc.addupdate_scatter(acc, [idx + BINS * LANES], ones)

    start(base, xb0, tb0, sx0, st0)

    def pair_body(i, carry):
        off0 = base + (2 * i) * CHE
        off1 = off0 + CHE
        start(off1, xb1, tb1, sx1, st1)
        wait(off0, xb0, tb0, sx0, st0)
        compute(xb0, tb0)

        @pl.when(i < NPAIR - 1)
        def _prefetch_even():
            start(off1 + CHE, xb0, tb0, sx0, st0)

        wait(off1, xb1, tb1, sx1, st1)
        compute(xb1, tb1)
        return carry

    lax.fori_loop(0, NPAIR, pair_body, 0)
    pltpu.sync_copy(acc, out_hbm.at[pl.ds(wid * NREP * ACCW, NREP * ACCW)])


def _tc_body(x_ref, t_ref, out_ref, acc_ref):
    i = pl.program_id(0)
    nsteps = pl.num_programs(0)

    def init_accs():
        return tuple(jnp.zeros((RCH, 128), jnp.float32) for _ in range(NACC))

    def load_accs():
        return tuple(acc_ref[k] for k in range(NACC))

    accs = jax.lax.cond(i == 0, init_accs, load_accs)

    def row_chunk(r, accs):
        accs = list(accs)
        for c0, w in _COLS:
            x = x_ref[pl.ds(r * RCH, RCH), pl.ds(c0, w)]
            t = t_ref[pl.ds(r * RCH, RCH), pl.ds(c0, w)]
            if w < 128:
                x = jnp.concatenate(
                    [x, jnp.full((RCH, 128 - w), jnp.inf, jnp.float32)], axis=1)
                t = jnp.concatenate(
                    [t, jnp.ones((RCH, 128 - w), jnp.float32)], axis=1)
            y = jnp.where(t > 0.5, -x, x)
            bce = jnp.maximum(y, 0.0) + jnp.log1p(jnp.exp(-jnp.abs(y)))
            accs[0] = accs[0] + bce
            for k in range(1, BINS):
                m = y >= _THRESH[k - 1]
                accs[2 * k - 1] = accs[2 * k - 1] + m.astype(jnp.float32)
                accs[2 * k] = accs[2 * k] + jnp.where(m, bce, 0.0)
        return tuple(accs)

    accs = jax.lax.fori_loop(0, BLK // RCH, row_chunk, accs)
    for k in range(NACC):
        acc_ref[k] = accs[k]

    @pl.when(i == nsteps - 1)
    def _finalize():
        out_ref[0] = jnp.sum(acc_ref[0])
        for k in range(1, BINS):
            out_ref[k] = jnp.sum(acc_ref[2 * k - 1])
            out_ref[9 + k] = jnp.sum(acc_ref[2 * k])


def _combine_body(a_ref, p_ref, out_ref):
    a = a_ref[...]
    s_tot = p_ref[0]
    ccum = [jnp.float32(TC_ROWS * C)]
    scum = [s_tot]
    for k in range(1, BINS):
        ccum.append(p_ref[k])
        scum.append(p_ref[9 + k])
    ccum.append(jnp.float32(0.0))
    scum.append(jnp.float32(0.0))
    n = jnp.float32(0.0)
    s = jnp.float32(0.0)
    for b in range(BINS):
        sb = jnp.sum(a[:, 16 * b:16 * b + 16]) + (scum[b] - scum[b + 1])
        cnt = jnp.sum(a[:, 160 + 16 * b:160 + 16 * b + 16]) \
            + (ccum[b] - ccum[b + 1])
        nonempty = cnt > 0.0
        n += jnp.where(nonempty, 1.0, 0.0)
        s += jnp.where(nonempty, sb / jnp.maximum(cnt, 1.0), 0.0)
    out_ref[0] = s / jnp.maximum(n, 1.0)


@jax.jit
def kernel(input, target):
    sc_accs = _sc_main(input[:SC_ROWS].reshape(-1),
                       target[:SC_ROWS].reshape(-1))
    tcp = pl.pallas_call(
        _tc_body,
        grid=(TC_ROWS // BLK,),
        in_specs=[
            pl.BlockSpec((BLK, C), lambda i: (i + SC_ROWS // BLK, 0)),
            pl.BlockSpec((BLK, C), lambda i: (i + SC_ROWS // BLK, 0)),
        ],
        out_specs=pl.BlockSpec(memory_space=pltpu.SMEM),
        out_shape=jax.ShapeDtypeStruct((2 * BINS - 1,), jnp.float32),
        scratch_shapes=[pltpu.VMEM((NACC, RCH, 128), jnp.float32)],
    )(input, target)
    out = pl.pallas_call(
        _combine_body,
        in_specs=[
            pl.BlockSpec((NW * NREP, ACCW), lambda: (0, 0)),
            pl.BlockSpec(memory_space=pltpu.SMEM),
        ],
        out_specs=pl.BlockSpec(memory_space=pltpu.SMEM),
        out_shape=jax.ShapeDtypeStruct((1,), jnp.float32),
    )(sc_accs.reshape(NW * NREP, ACCW), tcp)
    return out[0]


# speedup vs baseline: 3.2994x; 1.0001x over previous
import functools
import math

import jax
import jax.numpy as jnp
from jax import lax
from jax.experimental import pallas as pl
from jax.experimental.pallas import tpu as pltpu
from jax.experimental.pallas import tpu_sc as plsc

BINS = 10
B = 16384
C = 1000

SC_ROWS = 8192
TC_ROWS = B - SC_ROWS

SCN = SC_ROWS * C
NC, NS, LANES = 2, 16, 16
NW = NC * NS
PERW = SCN // NW
CHE = 16_000
NCHUNK = PERW // CHE
NPAIR = NCHUNK // 2
VPC = CHE // LANES
NREP = 4
ACCW = 2 * BINS * LANES

BLK = 256
RCH = 8
NACC = 2 * (BINS - 1) + 1
_THRESH = [math.log(k / (BINS - k)) for k in range(1, BINS)]
_COLS = [(ci * 128, 128) for ci in range(7)] + [(896, 104)]

_L1P = [9.975032552456131e-06, 0.999235483833275, -0.4902307234234118,
        0.2852726810905759, -0.13158182508875882, 0.030449004538667168]

_mesh = plsc.VectorSubcoreMesh(core_axis_name="c", subcore_axis_name="s")


@functools.partial(
    pl.kernel,
    out_type=jax.ShapeDtypeStruct((NW * NREP * ACCW,), jnp.float32),
    mesh=_mesh,
    scratch_types=[
        pltpu.VMEM((CHE,), jnp.float32),
        pltpu.VMEM((CHE,), jnp.float32),
        pltpu.VMEM((CHE,), jnp.float32),
        pltpu.VMEM((CHE,), jnp.float32),
        pltpu.VMEM((NREP * ACCW,), jnp.float32),
        pltpu.SemaphoreType.DMA,
        pltpu.SemaphoreType.DMA,
        pltpu.SemaphoreType.DMA,
        pltpu.SemaphoreType.DMA,
    ],
    compiler_params=pltpu.CompilerParams(needs_layout_passes=False),
)
def _sc_main(x_hbm, t_hbm, out_hbm, xb0, tb0, xb1, tb1, acc,
             sx0, st0, sx1, st1):
    wid = lax.axis_index("s") * NC + lax.axis_index("c")
    base = wid * PERW

    zeros = jnp.zeros((LANES,), jnp.float32)
    ones = jnp.ones((LANES,), jnp.float32)
    lane = lax.iota(jnp.int32, LANES)
    for k in range(NREP * 2 * BINS):
        acc[pl.ds(k * LANES, LANES)] = zeros

    def start(off, xb, tb, sx, st):
        pltpu.make_async_copy(x_hbm.at[pl.ds(off, CHE)], xb, sx).start()
        pltpu.make_async_copy(t_hbm.at[pl.ds(off, CHE)], tb, st).start()

    def wait(off, xb, tb, sx, st):
        pltpu.make_async_copy(x_hbm.at[pl.ds(off, CHE)], xb, sx).wait()
        pltpu.make_async_copy(t_hbm.at[pl.ds(off, CHE)], tb, st).wait()

    def compute(xb, tb):
        @plsc.parallel_loop(0, VPC, 1, unroll=8)
        def _vec(j):
            o = j * LANES
            x = xb[pl.ds(o, LANES)]
            t = tb[pl.ds(o, LANES)]
            y = jnp.where(t > 0.5, -x, x)
            u = jnp.exp(-jnp.abs(y))
            p = jnp.float32(_L1P[5])
            for c in (4, 3, 2, 1, 0):
                p = p * u + jnp.float32(_L1P[c])
            bce = jnp.maximum(y, 0.0) + p
            sig = jnp.where(y >= 0.0, 1.0, u) / (1.0 + u)
            bidx = jnp.minimum((sig * BINS).astype(jnp.int32), BINS - 1)
            rep = (j % NREP) * ACCW
            idx = bidx * LANES + lane + rep
            plsc.addupdate_scatter(acc, [idx], bce)
            plsc.addupdate_scatter(acc, [idx + BINS * LANES], ones)

    start(base, xb0, tb0, sx0, st0)

    def pair_body(i, carry):
        off0 = base + (2 * i) * CHE
        off1 = off0 + CHE
        start(off1, xb1, tb1, sx1, st1)
        wait(off0, xb0, tb0, sx0, st0)
        compute(xb0, tb0)

        @pl.when(i < NPAIR - 1)
        def _prefetch_even():
            start(off1 + CHE, xb0, tb0, sx0, st0)

        wait(off1, xb1, tb1, sx1, st1)
        compute(xb1, tb1)
        return carry

    lax.fori_loop(0, NPAIR, pair_body, 0)
    pltpu.sync_copy(acc, out_hbm.at[pl.ds(wid * NREP * ACCW, NREP * ACCW)])


def _tc_body(x_ref, t_ref, out_ref, acc_ref):
    i = pl.program_id(0)
    nsteps = pl.num_programs(0)

    def init_accs():
        return tuple(jnp.zeros((RCH, 128), jnp.float32) for _ in range(NACC))

    def load_accs():
        return tuple(acc_ref[k] for k in range(NACC))

    accs = jax.lax.cond(i == 0, init_accs, load_accs)

    def row_chunk(r, accs):
        accs = list(accs)
        for c0, w in _COLS:
            x = x_ref[pl.ds(r * RCH, RCH), pl.ds(c0, w)]
            t = t_ref[pl.ds(r * RCH, RCH), pl.ds(c0, w)]
            if w < 128:
                x = jnp.concatenate(
                    [x, jnp.full((RCH, 128 - w), jnp.inf, jnp.float32)], axis=1)
                t = jnp.concatenate(
                    [t, jnp.ones((RCH, 128 - w), jnp.float32)], axis=1)
            y = jnp.where(t > 0.5, -x, x)
            bce = jnp.maximum(y, 0.0) + jnp.log1p(jnp.exp(-jnp.abs(y)))
            accs[0] = accs[0] + bce
            for k in range(1, BINS):
                m = y >= _THRESH[k - 1]
                accs[2 * k - 1] = accs[2 * k - 1] + m.astype(jnp.float32)
                accs[2 * k] = accs[2 * k] + jnp.where(m, bce, 0.0)
        return tuple(accs)

    accs = jax.lax.fori_loop(0, BLK // RCH, row_chunk, accs)
    for k in range(NACC):
        acc_ref[k] = accs[k]

    @pl.when(i == nsteps - 1)
    def _finalize():
        out_ref[0] = jnp.sum(acc_ref[0])
        for k in range(1, BINS):
            out_ref[k] = jnp.sum(acc_ref[2 * k - 1])
            out_ref[9 + k] = jnp.sum(acc_ref[2 * k])


def _combine_body(a_ref, p_ref, out_ref):
    a = a_ref[...]
    s_tot = p_ref[0]
    ccum = [jnp.float32(TC_ROWS * C)]
    scum = [s_tot]
    for k in range(1, BINS):
        ccum.append(p_ref[k])
        scum.append(p_ref[9 + k])
    ccum.append(jnp.float32(0.0))
    scum.append(jnp.float32(0.0))
    n = jnp.float32(0.0)
    s = jnp.float32(0.0)
    for b in range(BINS):
        sb = jnp.sum(a[:, 16 * b:16 * b + 16]) + (scum[b] - scum[b + 1])
        cnt = jnp.sum(a[:, 160 + 16 * b:160 + 16 * b + 16]) \
            + (ccum[b] - ccum[b + 1])
        nonempty = cnt > 0.0
        n += jnp.where(nonempty, 1.0, 0.0)
        s += jnp.where(nonempty, sb / jnp.maximum(cnt, 1.0), 0.0)
    out_ref[0] = s / jnp.maximum(n, 1.0)


@jax.jit
def kernel(input, target):
    tcp = pl.pallas_call(
        _tc_body,
        grid=(TC_ROWS // BLK,),
        in_specs=[
            pl.BlockSpec((BLK, C), lambda i: (i + SC_ROWS // BLK, 0)),
            pl.BlockSpec((BLK, C), lambda i: (i + SC_ROWS // BLK, 0)),
        ],
        out_specs=pl.BlockSpec(memory_space=pltpu.SMEM),
        out_shape=jax.ShapeDtypeStruct((2 * BINS - 1,), jnp.float32),
        scratch_shapes=[pltpu.VMEM((NACC, RCH, 128), jnp.float32)],
    )(input, target)
    sc_accs = _sc_main(input[:SC_ROWS].reshape(-1),
                       target[:SC_ROWS].reshape(-1))
    out = pl.pallas_call(
        _combine_body,
        in_specs=[
            pl.BlockSpec((NW * NREP, ACCW), lambda: (0, 0)),
            pl.BlockSpec(memory_space=pltpu.SMEM),
        ],
        out_specs=pl.BlockSpec(memory_space=pltpu.SMEM),
        out_shape=jax.ShapeDtypeStruct((1,), jnp.float32),
    )(sc_accs.reshape(NW * NREP, ACCW), tcp)
    return out[0]


# speedup vs baseline: 3.3052x; 1.0018x over previous
import functools
import math

import jax
import jax.numpy as jnp
from jax import lax
from jax.experimental import pallas as pl
from jax.experimental.pallas import tpu as pltpu
from jax.experimental.pallas import tpu_sc as plsc

BINS = 10
B = 16384
C = 1000

SC_ROWS = 8192
TC_ROWS = B - SC_ROWS

SCN = SC_ROWS * C
NC, NS, LANES = 2, 16, 16
NW = NC * NS
PERW = SCN // NW
CHE = 16_000
NCHUNK = PERW // CHE
NPAIR = NCHUNK // 2
VPC = CHE // LANES
NREP = 4
ACCW = 2 * BINS * LANES

BLK = 256
RCH = 8
NACC = 2 * (BINS - 1) + 1
_THRESH = [math.log(k / (BINS - k)) for k in range(1, BINS)]
_COLS = [(ci * 128, 128) for ci in range(7)] + [(896, 104)]

_L1P = [9.975032552456131e-06, 0.999235483833275, -0.4902307234234118,
        0.2852726810905759, -0.13158182508875882, 0.030449004538667168]

_mesh = plsc.VectorSubcoreMesh(core_axis_name="c", subcore_axis_name="s")


@functools.partial(
    pl.kernel,
    out_type=jax.ShapeDtypeStruct((NW * NREP * ACCW,), jnp.float32),
    mesh=_mesh,
    scratch_types=[
        pltpu.VMEM((CHE,), jnp.float32),
        pltpu.VMEM((CHE,), jnp.float32),
        pltpu.VMEM((CHE,), jnp.float32),
        pltpu.VMEM((CHE,), jnp.float32),
        pltpu.VMEM((NREP * ACCW,), jnp.float32),
        pltpu.SemaphoreType.DMA,
        pltpu.SemaphoreType.DMA,
        pltpu.SemaphoreType.DMA,
        pltpu.SemaphoreType.DMA,
    ],
    compiler_params=pltpu.CompilerParams(needs_layout_passes=False),
)
def _sc_main(x_hbm, t_hbm, out_hbm, xb0, tb0, xb1, tb1, acc,
             sx0, st0, sx1, st1):
    wid = lax.axis_index("s") * NC + lax.axis_index("c")
    base = wid * PERW

    zeros = jnp.zeros((LANES,), jnp.float32)
    ones = jnp.ones((LANES,), jnp.float32)
    lane = lax.iota(jnp.int32, LANES)
    for k in range(NREP * 2 * BINS):
        acc[pl.ds(k * LANES, LANES)] = zeros

    def start(off, xb, tb, sx, st):
        pltpu.make_async_copy(x_hbm.at[pl.ds(off, CHE)], xb, sx).start()
        pltpu.make_async_copy(t_hbm.at[pl.ds(off, CHE)], tb, st).start()

    def wait(off, xb, tb, sx, st):
        pltpu.make_async_copy(x_hbm.at[pl.ds(off, CHE)], xb, sx).wait()
        pltpu.make_async_copy(t_hbm.at[pl.ds(off, CHE)], tb, st).wait()

    def compute(xb, tb):
        @plsc.parallel_loop(0, VPC, 1, unroll=8)
        def _vec(j):
            o = j * LANES
            x = xb[pl.ds(o, LANES)]
            t = tb[pl.ds(o, LANES)]
            y = jnp.where(t > 0.5, -x, x)
            u = jnp.exp(-jnp.abs(y))
            p = jnp.float32(_L1P[5])
            for c in (4, 3, 2, 1, 0):
                p = p * u + jnp.float32(_L1P[c])
            bce = jnp.maximum(y, 0.0) + p
            sig = jnp.where(y >= 0.0, 1.0, u) / (1.0 + u)
            bidx = jnp.minimum((sig * BINS).astype(jnp.int32), BINS - 1)
            rep = (j % NREP) * ACCW
            idx = bidx * LANES + lane + rep
            plsc.addupdate_scatter(acc, [idx], bce)
            plsc.addupdate_scatter(acc, [idx + BINS * LANES], ones)

    start(base, xb0, tb0, sx0, st0)

    def pair_body(i, carry):
        off0 = base + (2 * i) * CHE
        off1 = off0 + CHE
        start(off1, xb1, tb1, sx1, st1)
        wait(off0, xb0, tb0, sx0, st0)
        compute(xb0, tb0)

        @pl.when(i < NPAIR - 1)
        def _prefetch_even():
            start(off1 + CHE, xb0, tb0, sx0, st0)

        wait(off1, xb1, tb1, sx1, st1)
        compute(xb1, tb1)
        return carry

    lax.fori_loop(0, NPAIR, pair_body, 0)
    pltpu.sync_copy(acc, out_hbm.at[pl.ds(wid * NREP * ACCW, NREP * ACCW)])


def _tc_body(x_ref, t_ref, a_ref, out_ref, acc_ref):
    i = pl.program_id(0)
    nsteps = pl.num_programs(0)

    def init_accs():
        return tuple(jnp.zeros((RCH, 128), jnp.float32) for _ in range(NACC))

    def load_accs():
        return tuple(acc_ref[k] for k in range(NACC))

    accs = jax.lax.cond(i == 0, init_accs, load_accs)

    def row_chunk(r, accs):
        accs = list(accs)
        for c0, w in _COLS:
            x = x_ref[pl.ds(r * RCH, RCH), pl.ds(c0, w)]
            t = t_ref[pl.ds(r * RCH, RCH), pl.ds(c0, w)]
            if w < 128:
                x = jnp.concatenate(
                    [x, jnp.full((RCH, 128 - w), jnp.inf, jnp.float32)], axis=1)
                t = jnp.concatenate(
                    [t, jnp.ones((RCH, 128 - w), jnp.float32)], axis=1)
            y = jnp.where(t > 0.5, -x, x)
            bce = jnp.maximum(y, 0.0) + jnp.log1p(jnp.exp(-jnp.abs(y)))
            accs[0] = accs[0] + bce
            for k in range(1, BINS):
                m = y >= _THRESH[k - 1]
                accs[2 * k - 1] = accs[2 * k - 1] + m.astype(jnp.float32)
                accs[2 * k] = accs[2 * k] + jnp.where(m, bce, 0.0)
        return tuple(accs)

    accs = jax.lax.fori_loop(0, BLK // RCH, row_chunk, accs)
    for k in range(NACC):
        acc_ref[k] = accs[k]

    @pl.when(i == nsteps - 1)
    def _finalize():
        a = a_ref[...]
        ccum = [jnp.float32(TC_ROWS * C)]
        scum = [jnp.sum(acc_ref[0])]
        for k in range(1, BINS):
            ccum.append(jnp.sum(acc_ref[2 * k - 1]))
            scum.append(jnp.sum(acc_ref[2 * k]))
        ccum.append(jnp.float32(0.0))
        scum.append(jnp.float32(0.0))
        n = jnp.float32(0.0)
        s = jnp.float32(0.0)
        for b in range(BINS):
            sb = jnp.sum(a[:, 16 * b:16 * b + 16]) + (scum[b] - scum[b + 1])
            cnt = jnp.sum(a[:, 160 + 16 * b:160 + 16 * b + 16]) \
                + (ccum[b] - ccum[b + 1])
            nonempty = cnt > 0.0
            n += jnp.where(nonempty, 1.0, 0.0)
            s += jnp.where(nonempty, sb / jnp.maximum(cnt, 1.0), 0.0)
        out_ref[0] = s / jnp.maximum(n, 1.0)


@jax.jit
def kernel(input, target):
    sc_accs = _sc_main(input[:SC_ROWS].reshape(-1),
                       target[:SC_ROWS].reshape(-1))
    out = pl.pallas_call(
        _tc_body,
        grid=(TC_ROWS // BLK,),
        in_specs=[
            pl.BlockSpec((BLK, C), lambda i: (i + SC_ROWS // BLK, 0)),
            pl.BlockSpec((BLK, C), lambda i: (i + SC_ROWS // BLK, 0)),
            pl.BlockSpec((NW * NREP, ACCW), lambda i: (0, 0)),
        ],
        out_specs=pl.BlockSpec(memory_space=pltpu.SMEM),
        out_shape=jax.ShapeDtypeStruct((1,), jnp.float32),
        scratch_shapes=[pltpu.VMEM((NACC, RCH, 128), jnp.float32)],
    )(input, target, sc_accs.reshape(NW * NREP, ACCW))
    return out[0]
